# baseline (speedup 1.0000x reference)
import jax
import jax.numpy as jnp
from jax import lax
from jax.experimental import pallas as pl
from jax.experimental.pallas import tpu as pltpu
from jax.experimental.pallas import tpu_sc as plsc

N_ONTO = 10000
HID = 128
CONCAT = 3 * HID
N_NODE = 10000

NC, NS = 2, 16
NW = NC * NS
C = 128
ND = 256
TPAD = N_NODE + ND
NPAD = 10496
PER_TILE = NPAD // NS
E_PAD = 163840
CH = E_PAD // (NW * C)
HCH = 8
IPW = 320

_f32 = jnp.float32
_i32 = jnp.int32

_mesh = plsc.VectorSubcoreMesh(core_axis_name="c", subcore_axis_name="s",
                               num_cores=NC, num_subcores=NS)
_sc_params = pltpu.CompilerParams(needs_layout_passes=False)


def _mm(x, w, bias=None):
    M, K = x.shape
    N = w.shape[1]
    BM = 512
    G = (M + BM - 1) // BM

    def body(x_ref, w_ref, b_ref, o_ref):
        o_ref[...] = jnp.dot(x_ref[...], w_ref[...],
                             preferred_element_type=_f32) + b_ref[...]

    if bias is None:
        bias = jnp.zeros((N,), _f32)
    return pl.pallas_call(
        body,
        grid=(G,),
        in_specs=[
            pl.BlockSpec((BM, K), lambda i: (i, 0)),
            pl.BlockSpec((K, N), lambda i: (0, 0)),
            pl.BlockSpec((N,), lambda i: (0,)),
        ],
        out_specs=pl.BlockSpec((BM, N), lambda i: (i, 0)),
        out_shape=jax.ShapeDtypeStruct((M, N), _f32),
    )(x, w, bias)


def _fill_1d(ref, n16, vec16):
    for j in range(n16):
        ref[pl.ds(j * 16, 16)] = vec16


def _scale_rows(rows_v, w_ref, widx, n_rows, base_row=0):
    def scale(j, cc):
        for u in range(2):
            r = base_row + 2 * j + u
            wspl = plsc.load_gather(w_ref, widx(r))
            for k in range(HID // 16):
                rows_v[r, pl.ds(k * 16, 16)] = (
                    rows_v[r, pl.ds(k * 16, 16)] * wspl)
        return cc

    lax.fori_loop(0, n_rows // 2, scale, 0)


def _sc_gat_pass(h_list, a_flat, b_flat, s4, d4, zrows):
    NG = len(h_list)

    def body(*refs):
        h_hbms = refs[:NG]
        a_hbm, b_hbm, s_hbm, d_hbm, z_hbm, numer_out, denom_out = \
            refs[NG:NG + 7]
        (s_v, d_v, a_v, b_v, w_v, rows_v, z1_v, d1_v,
         numer_sp, denom_sp, sem, sem2) = refs[NG + 7:]
        c = lax.axis_index("c")
        t = lax.axis_index("s")
        wid = c * NS + t
        z16 = jnp.zeros((16,), _f32)
        _fill_1d(z1_v, PER_TILE // 16, z16)
        base = t * PER_TILE
        pltpu.sync_copy(z_hbm, numer_sp.at[pl.ds(base, PER_TILE)])
        pltpu.sync_copy(z1_v, denom_sp.at[pl.ds(base, PER_TILE)])
        plsc.subcore_barrier()
        for g in range(NG):
            pltpu.sync_copy(a_hbm.at[pl.ds(g * NPAD, TPAD)], a_v)
            pltpu.sync_copy(b_hbm.at[pl.ds(g * NPAD, TPAD)], b_v)
            h_hbm = h_hbms[g]
            for half in range(CH // HCH):
                pltpu.sync_copy(
                    s_hbm.at[g * NW + wid, pl.ds(half * HCH, HCH)], s_v)
                pltpu.sync_copy(
                    d_hbm.at[g * NW + wid, pl.ds(half * HCH, HCH)], d_v)

                def chunk(i, cc):
                    cpa = pltpu.make_async_copy(
                        h_hbm.at[s_v.at[i, pl.ds(0, C // 2)]],
                        rows_v.at[pl.ds(0, C // 2)], sem)
                    cpa.start()
                    cpb = pltpu.make_async_copy(
                        h_hbm.at[s_v.at[i, pl.ds(C // 2, C // 2)]],
                        rows_v.at[pl.ds(C // 2, C // 2)], sem2)
                    cpb.start()
                    for k in range(C // 16):
                        sv = s_v[i, pl.ds(k * 16, 16)]
                        dv = d_v[i, pl.ds(k * 16, 16)]
                        e = (plsc.load_gather(a_v, [sv])
                             + plsc.load_gather(b_v, [dv]))
                        e = jnp.where(e < 0, 0.2 * e, e)
                        w_v[pl.ds(k * 16, 16)] = jnp.exp(e)
                    widx = lambda r: [jnp.full((16,), r, _i32)]
                    cpa.wait()
                    _scale_rows(rows_v, w_v, widx, C // 2, 0)
                    cpb.wait()
                    _scale_rows(rows_v, w_v, widx, C // 2, C // 2)
                    pltpu.sync_copy(rows_v, numer_sp.at[d_v.at[i]], add=True)
                    pltpu.sync_copy(w_v, denom_sp.at[d_v.at[i]], add=True)
                    return cc

                lax.fori_loop(0, HCH, chunk, 0)
            plsc.subcore_barrier()
            pltpu.sync_copy(numer_sp.at[pl.ds(base, PER_TILE)],
                            numer_out.at[c, g, pl.ds(base, PER_TILE)])
            pltpu.sync_copy(denom_sp.at[pl.ds(base, PER_TILE)], d1_v)
            pltpu.sync_copy(
                d1_v,
                denom_out.at[pl.ds((c * NG + g) * NPAD + base, PER_TILE)])
            pltpu.sync_copy(z_hbm, numer_sp.at[pl.ds(base, PER_TILE)])
            pltpu.sync_copy(z1_v, denom_sp.at[pl.ds(base, PER_TILE)])
            plsc.subcore_barrier()

    return pl.kernel(
        body,
        out_type=[jax.ShapeDtypeStruct((NC, NG, NPAD, HID), _f32),
                  jax.ShapeDtypeStruct((NC * NG * NPAD,), _f32)],
        mesh=_mesh,
        compiler_params=_sc_params,
        scratch_types=[
            pltpu.VMEM((HCH, C), _i32),
            pltpu.VMEM((HCH, C), _i32),
            pltpu.VMEM((TPAD,), _f32),
            pltpu.VMEM((TPAD,), _f32),
            pltpu.VMEM((C,), _f32),
            pltpu.VMEM((C, HID), _f32),
            pltpu.VMEM((PER_TILE,), _f32),
            pltpu.VMEM((PER_TILE,), _f32),
            pltpu.VMEM_SHARED((NPAD, HID), _f32),
            pltpu.VMEM_SHARED((NPAD,), _f32),
            pltpu.SemaphoreType.DMA,
            pltpu.SemaphoreType.DMA,
        ],
    )(*h_list, a_flat, b_flat, s4, d4, zrows)


def _sc_gcn_pass(h_list, dinv_flat, s4, d4, zrows):
    NSD = len(h_list) // 3

    def body(*refs):
        h_hbms = refs[:3 * NSD]
        a_hbm, s_hbm, d_hbm, z_hbm, numer_out = refs[3 * NSD:3 * NSD + 5]
        (s_v, d_v, a_v, w_v, rows0, rows1,
         numer_sp, sem0, sem1, scsem) = refs[3 * NSD + 5:]
        c = lax.axis_index("c")
        t = lax.axis_index("s")
        wid = c * NS + t
        base = t * PER_TILE
        bufs = (rows0, rows1)
        sems = (sem0, sem1)
        pltpu.sync_copy(z_hbm, numer_sp.at[pl.ds(base, PER_TILE)])
        plsc.subcore_barrier()
        for side in range(NSD):
            pltpu.sync_copy(a_hbm.at[pl.ds(side * NPAD, TPAD)], a_v)
            row = side * NW + wid
            for cc in range(3):
                h_hbm = h_hbms[side * 3 + cc]

                def slab(sl, cc3):
                    off = pl.multiple_of(sl * HCH, HCH)
                    pltpu.sync_copy(s_hbm.at[row, pl.ds(off, HCH)], s_v)
                    pltpu.sync_copy(d_hbm.at[row, pl.ds(off, HCH)], d_v)

                    def wchunk(i, cc2):
                        for k in range(C // 16):
                            sv = s_v[i, pl.ds(k * 16, 16)]
                            dv = d_v[i, pl.ds(k * 16, 16)]
                            w_v[i, pl.ds(k * 16, 16)] = (
                                plsc.load_gather(a_v, [sv])
                                * plsc.load_gather(a_v, [dv]))
                        return cc2

                    lax.fori_loop(0, HCH, wchunk, 0)
                    pltpu.async_copy(h_hbm.at[s_v.at[0]], rows0, sem0)

                    def pair(j, cc2):
                        i0 = 2 * j

                        def do(i, buf, bsem, nbuf, nsem, first, last):
                            pltpu.make_async_copy(
                                h_hbm.at[s_v.at[i]], buf, bsem).wait()
                            _scale_rows(
                                buf, w_v,
                                lambda r: [jnp.full((16,), i, _i32),
                                           jnp.full((16,), r, _i32)], C)

                            @pl.when(jnp.logical_not(first))
                            def _():
                                pltpu.make_async_copy(
                                    nbuf, numer_sp.at[d_v.at[i - 1]],
                                    scsem).wait()

                            @pl.when(jnp.logical_not(last))
                            def _():
                                pltpu.async_copy(
                                    h_hbm.at[s_v.at[i + 1]], nbuf, nsem)
                            pltpu.make_async_copy(
                                buf, numer_sp.at[d_v.at[i]],
                                scsem).start(add=True)

                        do(i0, rows0, sem0, rows1, sem1, j == 0, False)
                        do(i0 + 1, rows1, sem1, rows0, sem0, False,
                           j == HCH // 2 - 1)
                        return cc2

                    lax.fori_loop(0, HCH // 2, pair, 0)
                    pltpu.make_async_copy(
                        rows1, numer_sp.at[d_v.at[HCH - 1]], scsem).wait()
                    return cc3

                lax.fori_loop(0, CH // HCH, slab, 0)
                plsc.subcore_barrier()
                pltpu.sync_copy(
                    numer_sp.at[pl.ds(base, PER_TILE)],
                    numer_out.at[c, side * 3 + cc, pl.ds(base, PER_TILE)])
                pltpu.sync_copy(z_hbm, numer_sp.at[pl.ds(base, PER_TILE)])
                plsc.subcore_barrier()

    return pl.kernel(
        body,
        out_type=jax.ShapeDtypeStruct((NC, 3 * NSD, NPAD, HID), _f32),
        mesh=_mesh,
        compiler_params=_sc_params,
        scratch_types=[
            pltpu.VMEM((HCH, C), _i32),
            pltpu.VMEM((HCH, C), _i32),
            pltpu.VMEM((TPAD,), _f32),
            pltpu.VMEM((HCH, C), _f32),
            pltpu.VMEM((C, HID), _f32),
            pltpu.VMEM((C, HID), _f32),
            pltpu.VMEM_SHARED((NPAD, HID), _f32),
            pltpu.SemaphoreType.DMA,
            pltpu.SemaphoreType.DMA,
            pltpu.SemaphoreType.DMA,
        ],
    )(*h_list, dinv_flat, s4, d4, zrows)


def _sc_gather(tbl, idx2):
    M = idx2.shape[0]
    NSD = M // NW

    def body(tbl_hbm, idx_hbm, out_hbm, idx_v, rows_v, sem):
        c = lax.axis_index("c")
        t = lax.axis_index("s")
        wid = c * NS + t
        for side in range(NSD):
            row = side * NW + wid
            pltpu.sync_copy(idx_hbm.at[row], idx_v)
            for g in range(IPW // 80):
                pltpu.async_copy(tbl_hbm.at[idx_v.at[pl.ds(g * 80, 80)]],
                                 rows_v, sem).wait()
                pltpu.sync_copy(rows_v,
                                out_hbm.at[pl.ds(row * IPW + g * 80, 80)])

    return pl.kernel(
        body,
        out_type=jax.ShapeDtypeStruct((M * IPW, CONCAT), _f32),
        mesh=_mesh,
        compiler_params=_sc_params,
        scratch_types=[
            pltpu.VMEM((IPW,), _i32),
            pltpu.VMEM((80, CONCAT), _f32),
            pltpu.SemaphoreType.DMA,
        ],
    )(tbl, idx2)


def _sc_deg(d4):
    NSD = d4.shape[0] // NW

    def body(d_hbm, deg_out, d_v, ones_v, z1_v, d1_v, deg_sp):
        c = lax.axis_index("c")
        t = lax.axis_index("s")
        wid = c * NS + t
        z16 = jnp.zeros((16,), _f32)
        o16 = jnp.ones((16,), _f32)
        _fill_1d(ones_v, C // 16, o16)
        _fill_1d(z1_v, PER_TILE // 16, z16)
        base = t * PER_TILE
        pltpu.sync_copy(z1_v, deg_sp.at[pl.ds(base, PER_TILE)])
        plsc.subcore_barrier()
        for side in range(NSD):
            pltpu.sync_copy(d_hbm.at[side * NW + wid], d_v)

            def chunk(i, cc):
                pltpu.sync_copy(ones_v, deg_sp.at[d_v.at[i]], add=True)
                return cc

            lax.fori_loop(0, CH, chunk, 0)
            plsc.subcore_barrier()
            pltpu.sync_copy(deg_sp.at[pl.ds(base, PER_TILE)], d1_v)
            pltpu.sync_copy(
                d1_v,
                deg_out.at[pl.ds((c * NSD + side) * NPAD + base, PER_TILE)])
            pltpu.sync_copy(z1_v, deg_sp.at[pl.ds(base, PER_TILE)])
            plsc.subcore_barrier()

    return pl.kernel(
        body,
        out_type=jax.ShapeDtypeStruct((NC * NSD * NPAD,), _f32),
        mesh=_mesh,
        compiler_params=_sc_params,
        scratch_types=[
            pltpu.VMEM((CH, C), _i32),
            pltpu.VMEM((C,), _f32),
            pltpu.VMEM((PER_TILE,), _f32),
            pltpu.VMEM((PER_TILE,), _f32),
            pltpu.VMEM_SHARED((NPAD,), _f32),
        ],
    )(d4)


def _pad_edges(s, d):
    npad = E_PAD - s.shape[0]
    ar = jnp.arange(npad, dtype=_i32) % ND
    s3 = jnp.concatenate([s, ar]).reshape(NW, CH, C)
    d3 = jnp.concatenate([d, N_NODE + ar]).reshape(NW, CH, C)
    return s3, d3


def _pad_nodes(x):
    return jnp.pad(x, (0, NPAD - N_NODE))


def kernel(params, left_x, right_x, left_graph_index, right_graph_index,
           left_x_batch, right_x_batch, left_diag_cnt, right_diag_cnt,
           edges1_diag, edges2_diag, edges1_proce, edges2_proce,
           edges1_atc, edges2_atc):
    p = params
    names = ["diag", "proce", "atc"]
    edges = {"diag": (edges1_diag, edges2_diag),
             "proce": (edges1_proce, edges2_proce),
             "atc": (edges1_atc, edges2_atc)}
    zrows = jnp.zeros((PER_TILE, HID), _f32)

    Wext, bias = {}, {}
    for name in names:
        W = p["gat_" + name + "_W"]
        wsrc = W @ p["gat_" + name + "_asrc"]
        wdst = W @ p["gat_" + name + "_adst"]
        Wext[name] = jnp.concatenate([W, wsrc[:, None], wdst[:, None]],
                                     axis=1)
        bias[name] = p["gat_" + name + "_b"]

    xs = {name: p["emb_" + name] for name in names}
    for layer in range(2):
        hs_all, hd_all, h_all, sl, dl = {}, {}, [], [], []
        for name in names:
            hext = _mm(xs[name], Wext[name])
            h_all.append(hext[:, :HID])
            hs_all[name] = hext[:, HID]
            hd_all[name] = hext[:, HID + 1]
            e = edges[name][layer]
            s3, d3 = _pad_edges(e[0], e[1])
            sl.append(s3)
            dl.append(d3)
        a_flat = jnp.concatenate([_pad_nodes(hs_all[n]) for n in names])
        b_flat = jnp.concatenate([_pad_nodes(hd_all[n]) for n in names])
        numer, denom = _sc_gat_pass(h_all, a_flat, b_flat,
                                    jnp.concatenate(sl), jnp.concatenate(dl),
                                    zrows)
        denom = denom.reshape(NC, 3, NPAD)
        for g, name in enumerate(names):
            nm = (numer[0, g] + numer[1, g])[:N_NODE]
            dn = (denom[0, g] + denom[1, g])[:N_NODE]
            el = hs_all[name] + hd_all[name]
            wl = jnp.exp(jnp.where(el < 0, 0.2 * el, el))
            xs[name] = ((nm + h_all[g] * wl[:, None])
                        / (dn + wl)[:, None] + bias[name])

    all_emb = jnp.concatenate([xs[n] for n in names], axis=0)
    descW = _mm(p["desc_emb"], p["Wd"], p["bd"])
    imgW = _mm(p["img_emb"], p["Wi"], p["bi"])
    x_all = jnp.concatenate([all_emb, descW, imgW], axis=1)

    npad_i = NW * IPW - N_NODE
    pad_i = jnp.arange(npad_i, dtype=_i32) % ND
    idx2 = jnp.concatenate(
        [jnp.concatenate([left_x[:, 0], pad_i]),
         jnp.concatenate([right_x[:, 0], pad_i])]).reshape(2 * NW, IPW)
    xg = _sc_gather(x_all, idx2)
    x_sides = [xg[:N_NODE], xg[NW * IPW:NW * IPW + N_NODE]]

    sL, dL = _pad_edges(left_graph_index[0], left_graph_index[1])
    sR, dR = _pad_edges(right_graph_index[0], right_graph_index[1])
    s4 = jnp.concatenate([sL, sR])
    d4 = jnp.concatenate([dL, dR])
    deg = _sc_deg(d4).reshape(NC, 2, NPAD)
    dinvs = []
    for side in range(2):
        dg = 1.0 + (deg[0, side] + deg[1, side])[:N_NODE]
        dinvs.append(lax.rsqrt(jnp.clip(dg, 1.0, None)))
    dinv_flat = jnp.concatenate([_pad_nodes(dv) for dv in dinvs])

    for layer, (W, b) in enumerate([(p["gcn1_W"], p["gcn1_b"]),
                                    (p["gcn2_W"], p["gcn2_b"])]):
        hs = [_mm(x_sides[side], W) for side in range(2)]
        h_list = [hs[side][:, cc * HID:(cc + 1) * HID]
                  for side in range(2) for cc in range(3)]
        numer = _sc_gcn_pass(h_list, dinv_flat, s4, d4, zrows)
        numer = numer[0] + numer[1]
        for side in range(2):
            agg = jnp.concatenate([numer[side * 3 + cc, :N_NODE]
                                   for cc in range(3)], axis=1)
            out = agg + hs[side] * (dinvs[side] * dinvs[side])[:, None] + b
            if layer == 0:
                out = jnp.where(out > 0, out, jnp.expm1(out))
            x_sides[side] = out

    return x_sides[0], x_sides[1]


# speedup vs baseline: 1.1527x; 1.1527x over previous
import jax
import jax.numpy as jnp
from jax import lax
from jax.experimental import pallas as pl
from jax.experimental.pallas import tpu as pltpu
from jax.experimental.pallas import tpu_sc as plsc

N_ONTO = 10000
HID = 128
CONCAT = 3 * HID
N_NODE = 10000

NC, NS = 2, 16
NW = NC * NS
C = 128
ND = 256
TPAD = N_NODE + ND
NPAD = 10496
PER_TILE = NPAD // NS
E_PAD = 163840
CH = E_PAD // (NW * C)
HCH = 8
IPW = 320

_f32 = jnp.float32
_i32 = jnp.int32

_mesh = plsc.VectorSubcoreMesh(core_axis_name="c", subcore_axis_name="s",
                               num_cores=NC, num_subcores=NS)
_sc_params = pltpu.CompilerParams(needs_layout_passes=False)


def _mm(x, w, bias=None):
    M, K = x.shape
    N = w.shape[1]
    BM = 512
    G = (M + BM - 1) // BM

    def body(x_ref, w_ref, b_ref, o_ref):
        o_ref[...] = jnp.dot(x_ref[...], w_ref[...],
                             preferred_element_type=_f32) + b_ref[...]

    if bias is None:
        bias = jnp.zeros((N,), _f32)
    return pl.pallas_call(
        body,
        grid=(G,),
        in_specs=[
            pl.BlockSpec((BM, K), lambda i: (i, 0)),
            pl.BlockSpec((K, N), lambda i: (0, 0)),
            pl.BlockSpec((N,), lambda i: (0,)),
        ],
        out_specs=pl.BlockSpec((BM, N), lambda i: (i, 0)),
        out_shape=jax.ShapeDtypeStruct((M, N), _f32),
    )(x, w, bias)


def _fill_1d(ref, n16, vec16):
    for j in range(n16):
        ref[pl.ds(j * 16, 16)] = vec16


def _scale_rows(rows_v, w_ref, widx, n_rows, base_row=0):
    def scale(j, cc):
        for u in range(2):
            r = base_row + 2 * j + u
            wspl = plsc.load_gather(w_ref, widx(r))
            for k in range(HID // 16):
                rows_v[r, pl.ds(k * 16, 16)] = (
                    rows_v[r, pl.ds(k * 16, 16)] * wspl)
        return cc

    lax.fori_loop(0, n_rows // 2, scale, 0)


def _sc_gat_pass(h_list, a_flat, b_flat, s4, d4, zrows):
    NG = len(h_list)

    def body(*refs):
        h_hbms = refs[:NG]
        a_hbm, b_hbm, s_hbm, d_hbm, z_hbm, numer_out, denom_out = \
            refs[NG:NG + 7]
        (s_v, d_v, a_v, b_v, w_v, rows_v, z1_v, d1_v,
         numer_sp, denom_sp, sem, sem2) = refs[NG + 7:]
        c = lax.axis_index("c")
        t = lax.axis_index("s")
        wid = c * NS + t
        z16 = jnp.zeros((16,), _f32)
        _fill_1d(z1_v, PER_TILE // 16, z16)
        base = t * PER_TILE
        pltpu.sync_copy(z_hbm, numer_sp.at[pl.ds(base, PER_TILE)])
        pltpu.sync_copy(z1_v, denom_sp.at[pl.ds(base, PER_TILE)])
        plsc.subcore_barrier()
        for g in range(NG):
            pltpu.sync_copy(a_hbm.at[pl.ds(g * NPAD, TPAD)], a_v)
            pltpu.sync_copy(b_hbm.at[pl.ds(g * NPAD, TPAD)], b_v)
            h_hbm = h_hbms[g]
            for half in range(CH // HCH):
                pltpu.sync_copy(
                    s_hbm.at[g * NW + wid, pl.ds(half * HCH, HCH)], s_v)
                pltpu.sync_copy(
                    d_hbm.at[g * NW + wid, pl.ds(half * HCH, HCH)], d_v)

                def chunk(i, cc):
                    cpa = pltpu.make_async_copy(
                        h_hbm.at[s_v.at[i, pl.ds(0, C // 2)]],
                        rows_v.at[pl.ds(0, C // 2)], sem)
                    cpa.start()
                    cpb = pltpu.make_async_copy(
                        h_hbm.at[s_v.at[i, pl.ds(C // 2, C // 2)]],
                        rows_v.at[pl.ds(C // 2, C // 2)], sem2)
                    cpb.start()
                    for k in range(C // 16):
                        sv = s_v[i, pl.ds(k * 16, 16)]
                        dv = d_v[i, pl.ds(k * 16, 16)]
                        e = (plsc.load_gather(a_v, [sv])
                             + plsc.load_gather(b_v, [dv]))
                        e = jnp.where(e < 0, 0.2 * e, e)
                        w_v[pl.ds(k * 16, 16)] = jnp.exp(e)
                    widx = lambda r: [jnp.full((16,), r, _i32)]
                    cpa.wait()
                    _scale_rows(rows_v, w_v, widx, C // 2, 0)
                    cpb.wait()
                    _scale_rows(rows_v, w_v, widx, C // 2, C // 2)
                    pltpu.sync_copy(rows_v, numer_sp.at[d_v.at[i]], add=True)
                    pltpu.sync_copy(w_v, denom_sp.at[d_v.at[i]], add=True)
                    return cc

                lax.fori_loop(0, HCH, chunk, 0)
            plsc.subcore_barrier()
            pltpu.sync_copy(numer_sp.at[pl.ds(base, PER_TILE)],
                            numer_out.at[c, g, pl.ds(base, PER_TILE)])
            pltpu.sync_copy(denom_sp.at[pl.ds(base, PER_TILE)], d1_v)
            pltpu.sync_copy(
                d1_v,
                denom_out.at[pl.ds((c * NG + g) * NPAD + base, PER_TILE)])
            pltpu.sync_copy(z_hbm, numer_sp.at[pl.ds(base, PER_TILE)])
            pltpu.sync_copy(z1_v, denom_sp.at[pl.ds(base, PER_TILE)])
            plsc.subcore_barrier()

    return pl.kernel(
        body,
        out_type=[jax.ShapeDtypeStruct((NC, NG, NPAD, HID), _f32),
                  jax.ShapeDtypeStruct((NC * NG * NPAD,), _f32)],
        mesh=_mesh,
        compiler_params=_sc_params,
        scratch_types=[
            pltpu.VMEM((HCH, C), _i32),
            pltpu.VMEM((HCH, C), _i32),
            pltpu.VMEM((TPAD,), _f32),
            pltpu.VMEM((TPAD,), _f32),
            pltpu.VMEM((C,), _f32),
            pltpu.VMEM((C, HID), _f32),
            pltpu.VMEM((PER_TILE,), _f32),
            pltpu.VMEM((PER_TILE,), _f32),
            pltpu.VMEM_SHARED((NPAD, HID), _f32),
            pltpu.VMEM_SHARED((NPAD,), _f32),
            pltpu.SemaphoreType.DMA,
            pltpu.SemaphoreType.DMA,
        ],
    )(*h_list, a_flat, b_flat, s4, d4, zrows)


CH2 = E_PAD // (NS * C)


def _sc_gcn_pass(hcat, dinv_flat, s4, d4, zrows):
    def body(*refs):
        (h_hbm, a_hbm, s_hbm, d_hbm, z_hbm, numer_out,
         s_v, d_v, a_v, w_v, rows0, rows1,
         numer_sp, sem0, sem1, scsem) = refs
        c = lax.axis_index("c")
        t = lax.axis_index("s")
        base = t * PER_TILE
        pltpu.sync_copy(z_hbm, numer_sp.at[pl.ds(base, PER_TILE)])
        pltpu.sync_copy(a_hbm.at[pl.ds(c * NPAD, TPAD)], a_v)
        plsc.subcore_barrier()
        if True:
            row = c * NS + t
            for cc in range(3):
                coff = jnp.full((16,), (c * 3 + cc) * N_NODE, _i32)

                def slab(sl, cc3):
                    off = pl.multiple_of(sl * HCH, HCH)
                    pltpu.sync_copy(s_hbm.at[row, pl.ds(off, HCH)], s_v)
                    pltpu.sync_copy(d_hbm.at[row, pl.ds(off, HCH)], d_v)

                    def wchunk(i, cc2):
                        for k in range(C // 16):
                            sv = s_v[i, pl.ds(k * 16, 16)]
                            dv = d_v[i, pl.ds(k * 16, 16)]
                            w_v[i, pl.ds(k * 16, 16)] = (
                                plsc.load_gather(a_v, [sv])
                                * plsc.load_gather(a_v, [dv]))
                            s_v[i, pl.ds(k * 16, 16)] = sv + coff
                        return cc2

                    lax.fori_loop(0, HCH, wchunk, 0)
                    pltpu.async_copy(h_hbm.at[s_v.at[0]], rows0, sem0)

                    def pair(j, cc2):
                        i0 = 2 * j

                        def do(i, buf, bsem, nbuf, nsem, last):
                            pltpu.make_async_copy(
                                h_hbm.at[s_v.at[i]], buf, bsem).wait()

                            @pl.when(jnp.logical_not(last))
                            def _():
                                pltpu.async_copy(
                                    h_hbm.at[s_v.at[i + 1]], nbuf, nsem)
                            _scale_rows(
                                buf, w_v,
                                lambda r: [jnp.full((16,), i, _i32),
                                           jnp.full((16,), r, _i32)], C)
                            pltpu.sync_copy(buf, numer_sp.at[d_v.at[i]],
                                            add=True)

                        do(i0, rows0, sem0, rows1, sem1, False)
                        do(i0 + 1, rows1, sem1, rows0, sem0,
                           j == HCH // 2 - 1)
                        return cc2

                    lax.fori_loop(0, HCH // 2, pair, 0)
                    return cc3

                lax.fori_loop(0, CH2 // HCH, slab, 0)
                plsc.subcore_barrier()
                pltpu.sync_copy(
                    numer_sp.at[pl.ds(base, PER_TILE)],
                    numer_out.at[c, cc, pl.ds(base, PER_TILE)])
                pltpu.sync_copy(z_hbm, numer_sp.at[pl.ds(base, PER_TILE)])
                plsc.subcore_barrier()

    return pl.kernel(
        body,
        out_type=jax.ShapeDtypeStruct((NC, 3, NPAD, HID), _f32),
        mesh=_mesh,
        compiler_params=_sc_params,
        scratch_types=[
            pltpu.VMEM((HCH, C), _i32),
            pltpu.VMEM((HCH, C), _i32),
            pltpu.VMEM((TPAD,), _f32),
            pltpu.VMEM((HCH, C), _f32),
            pltpu.VMEM((C, HID), _f32),
            pltpu.VMEM((C, HID), _f32),
            pltpu.VMEM_SHARED((NPAD, HID), _f32),
            pltpu.SemaphoreType.DMA,
            pltpu.SemaphoreType.DMA,
            pltpu.SemaphoreType.DMA,
        ],
    )(hcat, dinv_flat, s4, d4, zrows)


def _sc_gather(tbl, idx2):
    M = idx2.shape[0]
    NSD = M // NW

    def body(tbl_hbm, idx_hbm, out_hbm, idx_v, rows_v, sem):
        c = lax.axis_index("c")
        t = lax.axis_index("s")
        wid = c * NS + t
        for side in range(NSD):
            row = side * NW + wid
            pltpu.sync_copy(idx_hbm.at[row], idx_v)
            for g in range(IPW // 80):
                pltpu.async_copy(tbl_hbm.at[idx_v.at[pl.ds(g * 80, 80)]],
                                 rows_v, sem).wait()
                pltpu.sync_copy(rows_v,
                                out_hbm.at[pl.ds(row * IPW + g * 80, 80)])

    return pl.kernel(
        body,
        out_type=jax.ShapeDtypeStruct((M * IPW, CONCAT), _f32),
        mesh=_mesh,
        compiler_params=_sc_params,
        scratch_types=[
            pltpu.VMEM((IPW,), _i32),
            pltpu.VMEM((80, CONCAT), _f32),
            pltpu.SemaphoreType.DMA,
        ],
    )(tbl, idx2)


def _sc_deg(d4):
    NSD = d4.shape[0] // NW

    def body(d_hbm, deg_out, d_v, ones_v, z1_v, d1_v, deg_sp):
        c = lax.axis_index("c")
        t = lax.axis_index("s")
        wid = c * NS + t
        z16 = jnp.zeros((16,), _f32)
        o16 = jnp.ones((16,), _f32)
        _fill_1d(ones_v, C // 16, o16)
        _fill_1d(z1_v, PER_TILE // 16, z16)
        base = t * PER_TILE
        pltpu.sync_copy(z1_v, deg_sp.at[pl.ds(base, PER_TILE)])
        plsc.subcore_barrier()
        for side in range(NSD):
            pltpu.sync_copy(d_hbm.at[side * NW + wid], d_v)

            def chunk(i, cc):
                pltpu.sync_copy(ones_v, deg_sp.at[d_v.at[i]], add=True)
                return cc

            lax.fori_loop(0, CH, chunk, 0)
            plsc.subcore_barrier()
            pltpu.sync_copy(deg_sp.at[pl.ds(base, PER_TILE)], d1_v)
            pltpu.sync_copy(
                d1_v,
                deg_out.at[pl.ds((c * NSD + side) * NPAD + base, PER_TILE)])
            pltpu.sync_copy(z1_v, deg_sp.at[pl.ds(base, PER_TILE)])
            plsc.subcore_barrier()

    return pl.kernel(
        body,
        out_type=jax.ShapeDtypeStruct((NC * NSD * NPAD,), _f32),
        mesh=_mesh,
        compiler_params=_sc_params,
        scratch_types=[
            pltpu.VMEM((CH, C), _i32),
            pltpu.VMEM((C,), _f32),
            pltpu.VMEM((PER_TILE,), _f32),
            pltpu.VMEM((PER_TILE,), _f32),
            pltpu.VMEM_SHARED((NPAD,), _f32),
        ],
    )(d4)


def _pad_edges(s, d, shape=(NW, CH, C)):
    npad = E_PAD - s.shape[0]
    ar = jnp.arange(npad, dtype=_i32) % ND
    s3 = jnp.concatenate([s, ar]).reshape(shape)
    d3 = jnp.concatenate([d, N_NODE + ar]).reshape(shape)
    return s3, d3


def _pad_nodes(x):
    return jnp.pad(x, (0, NPAD - N_NODE))


def kernel(params, left_x, right_x, left_graph_index, right_graph_index,
           left_x_batch, right_x_batch, left_diag_cnt, right_diag_cnt,
           edges1_diag, edges2_diag, edges1_proce, edges2_proce,
           edges1_atc, edges2_atc):
    p = params
    names = ["diag", "proce", "atc"]
    edges = {"diag": (edges1_diag, edges2_diag),
             "proce": (edges1_proce, edges2_proce),
             "atc": (edges1_atc, edges2_atc)}
    zrows = jnp.zeros((PER_TILE, HID), _f32)

    Wext, bias = {}, {}
    for name in names:
        W = p["gat_" + name + "_W"]
        wsrc = W @ p["gat_" + name + "_asrc"]
        wdst = W @ p["gat_" + name + "_adst"]
        Wext[name] = jnp.concatenate([W, wsrc[:, None], wdst[:, None]],
                                     axis=1)
        bias[name] = p["gat_" + name + "_b"]

    xs = {name: p["emb_" + name] for name in names}
    for layer in range(2):
        hs_all, hd_all, h_all, sl, dl = {}, {}, [], [], []
        for name in names:
            hext = _mm(xs[name], Wext[name])
            h_all.append(hext[:, :HID])
            hs_all[name] = hext[:, HID]
            hd_all[name] = hext[:, HID + 1]
            e = edges[name][layer]
            s3, d3 = _pad_edges(e[0], e[1])
            sl.append(s3)
            dl.append(d3)
        a_flat = jnp.concatenate([_pad_nodes(hs_all[n]) for n in names])
        b_flat = jnp.concatenate([_pad_nodes(hd_all[n]) for n in names])
        numer, denom = _sc_gat_pass(h_all, a_flat, b_flat,
                                    jnp.concatenate(sl), jnp.concatenate(dl),
                                    zrows)
        denom = denom.reshape(NC, 3, NPAD)
        for g, name in enumerate(names):
            nm = (numer[0, g] + numer[1, g])[:N_NODE]
            dn = (denom[0, g] + denom[1, g])[:N_NODE]
            el = hs_all[name] + hd_all[name]
            wl = jnp.exp(jnp.where(el < 0, 0.2 * el, el))
            xs[name] = ((nm + h_all[g] * wl[:, None])
                        / (dn + wl)[:, None] + bias[name])

    all_emb = jnp.concatenate([xs[n] for n in names], axis=0)
    descW = _mm(p["desc_emb"], p["Wd"], p["bd"])
    imgW = _mm(p["img_emb"], p["Wi"], p["bi"])
    x_all = jnp.concatenate([all_emb, descW, imgW], axis=1)

    npad_i = NW * IPW - N_NODE
    pad_i = jnp.arange(npad_i, dtype=_i32) % ND
    idx2 = jnp.concatenate(
        [jnp.concatenate([left_x[:, 0], pad_i]),
         jnp.concatenate([right_x[:, 0], pad_i])]).reshape(2 * NW, IPW)
    xg = _sc_gather(x_all, idx2)
    x_sides = [xg[:N_NODE], xg[NW * IPW:NW * IPW + N_NODE]]

    sL, dL = _pad_edges(left_graph_index[0], left_graph_index[1])
    sR, dR = _pad_edges(right_graph_index[0], right_graph_index[1])
    d4 = jnp.concatenate([dL, dR])
    sLg, dLg = _pad_edges(left_graph_index[0], left_graph_index[1],
                          (NS, CH2, C))
    sRg, dRg = _pad_edges(right_graph_index[0], right_graph_index[1],
                          (NS, CH2, C))
    s4g = jnp.concatenate([sLg, sRg])
    d4g = jnp.concatenate([dLg, dRg])
    deg = _sc_deg(d4).reshape(NC, 2, NPAD)
    dinvs = []
    for side in range(2):
        dg = 1.0 + (deg[0, side] + deg[1, side])[:N_NODE]
        dinvs.append(lax.rsqrt(jnp.clip(dg, 1.0, None)))
    dinv_flat = jnp.concatenate([_pad_nodes(dv) for dv in dinvs])

    for layer, (W, b) in enumerate([(p["gcn1_W"], p["gcn1_b"]),
                                    (p["gcn2_W"], p["gcn2_b"])]):
        hs = [_mm(x_sides[side], W) for side in range(2)]
        hcat = jnp.concatenate(
            [hs[side][:, cc * HID:(cc + 1) * HID]
             for side in range(2) for cc in range(3)], axis=0)
        numer = _sc_gcn_pass(hcat, dinv_flat, s4g, d4g, zrows)
        for side in range(2):
            agg = jnp.concatenate([numer[side, cc, :N_NODE]
                                   for cc in range(3)], axis=1)
            out = agg + hs[side] * (dinvs[side] * dinvs[side])[:, None] + b
            if layer == 0:
                out = jnp.where(out > 0, out, jnp.expm1(out))
            x_sides[side] = out

    return x_sides[0], x_sides[1]


# speedup vs baseline: 1.2498x; 1.0843x over previous
import jax
import jax.numpy as jnp
from jax import lax
from jax.experimental import pallas as pl
from jax.experimental.pallas import tpu as pltpu
from jax.experimental.pallas import tpu_sc as plsc

N_ONTO = 10000
HID = 128
CONCAT = 3 * HID
N_NODE = 10000

NC, NS = 2, 16
NW = NC * NS
C = 128
ND = 256
TPAD = N_NODE + ND
NPAD = 10496
PER_TILE = NPAD // NS
E_PAD = 163840
CH = E_PAD // (NW * C)
HCH = 8
IPW = 320

_f32 = jnp.float32
_i32 = jnp.int32

_mesh = plsc.VectorSubcoreMesh(core_axis_name="c", subcore_axis_name="s",
                               num_cores=NC, num_subcores=NS)
_sc_params = pltpu.CompilerParams(needs_layout_passes=False)


def _mm(x, w, bias=None):
    M, K = x.shape
    N = w.shape[1]
    BM = 512
    G = (M + BM - 1) // BM

    def body(x_ref, w_ref, b_ref, o_ref):
        o_ref[...] = jnp.dot(x_ref[...], w_ref[...],
                             preferred_element_type=_f32) + b_ref[...]

    if bias is None:
        bias = jnp.zeros((N,), _f32)
    return pl.pallas_call(
        body,
        grid=(G,),
        in_specs=[
            pl.BlockSpec((BM, K), lambda i: (i, 0)),
            pl.BlockSpec((K, N), lambda i: (0, 0)),
            pl.BlockSpec((N,), lambda i: (0,)),
        ],
        out_specs=pl.BlockSpec((BM, N), lambda i: (i, 0)),
        out_shape=jax.ShapeDtypeStruct((M, N), _f32),
    )(x, w, bias)


def _fill_1d(ref, n16, vec16):
    for j in range(n16):
        ref[pl.ds(j * 16, 16)] = vec16


def _scale_rows(rows_v, w_ref, widx, n_rows, base_row=0):
    def scale(j, cc):
        for u in range(2):
            r = base_row + 2 * j + u
            wspl = plsc.load_gather(w_ref, widx(r))
            for k in range(HID // 16):
                rows_v[r, pl.ds(k * 16, 16)] = (
                    rows_v[r, pl.ds(k * 16, 16)] * wspl)
        return cc

    lax.fori_loop(0, n_rows // 2, scale, 0)


def _sc_gat_wpass(a_flat, b_flat, s4, d4):
    NG = s4.shape[0] // NW

    def body(a_hbm, b_hbm, s_hbm, d_hbm, w_out, denom_out,
             s_v, d_v, a_v, b_v, w_v, z1_v, d1_v, denom_sp):
        c = lax.axis_index("c")
        t = lax.axis_index("s")
        wid = c * NS + t
        z16 = jnp.zeros((16,), _f32)
        _fill_1d(z1_v, PER_TILE // 16, z16)
        base = t * PER_TILE
        pltpu.sync_copy(z1_v, denom_sp.at[pl.ds(base, PER_TILE)])
        plsc.subcore_barrier()
        for g in range(NG):
            pltpu.sync_copy(a_hbm.at[pl.ds(g * NPAD, TPAD)], a_v)
            pltpu.sync_copy(b_hbm.at[pl.ds(g * NPAD, TPAD)], b_v)
            pltpu.sync_copy(s_hbm.at[g * NW + wid], s_v)
            pltpu.sync_copy(d_hbm.at[g * NW + wid], d_v)

            def chunk(i, cc):
                for k in range(C // 16):
                    sv = s_v[i, pl.ds(k * 16, 16)]
                    dv = d_v[i, pl.ds(k * 16, 16)]
                    e = (plsc.load_gather(a_v, [sv])
                         + plsc.load_gather(b_v, [dv]))
                    e = jnp.where(e < 0, 0.2 * e, e)
                    w_v[i, pl.ds(k * 16, 16)] = jnp.exp(e)
                pltpu.sync_copy(w_v.at[i], denom_sp.at[d_v.at[i]], add=True)
                return cc

            lax.fori_loop(0, CH, chunk, 0)
            pltpu.sync_copy(w_v, w_out.at[g * NW + wid])
            plsc.subcore_barrier()
            pltpu.sync_copy(denom_sp.at[pl.ds(base, PER_TILE)], d1_v)
            pltpu.sync_copy(
                d1_v,
                denom_out.at[pl.ds((c * NG + g) * NPAD + base, PER_TILE)])
            pltpu.sync_copy(z1_v, denom_sp.at[pl.ds(base, PER_TILE)])
            plsc.subcore_barrier()

    return pl.kernel(
        body,
        out_type=[jax.ShapeDtypeStruct((NG * NW, CH, C), _f32),
                  jax.ShapeDtypeStruct((NC * NG * NPAD,), _f32)],
        mesh=_mesh,
        compiler_params=_sc_params,
        scratch_types=[
            pltpu.VMEM((CH, C), _i32),
            pltpu.VMEM((CH, C), _i32),
            pltpu.VMEM((TPAD,), _f32),
            pltpu.VMEM((TPAD,), _f32),
            pltpu.VMEM((CH, C), _f32),
            pltpu.VMEM((PER_TILE,), _f32),
            pltpu.VMEM((PER_TILE,), _f32),
            pltpu.VMEM_SHARED((NPAD,), _f32),
        ],
    )(a_flat, b_flat, s4, d4)


def _sc_gat_rows(h_list, w_flat, s4, d4, zrows):
    NG = len(h_list)

    def body(*refs):
        h_hbms = refs[:NG]
        w_hbm, s_hbm, d_hbm, z_hbm, numer_out = refs[NG:NG + 5]
        (s_v, d_v, w_v, rows0, rows1, numer_sp, sem0, sem1) = refs[NG + 5:]
        c = lax.axis_index("c")
        t = lax.axis_index("s")
        wid = c * NS + t
        base = t * PER_TILE
        pltpu.sync_copy(z_hbm, numer_sp.at[pl.ds(base, PER_TILE)])
        plsc.subcore_barrier()
        for g in range(NG):
            h_hbm = h_hbms[g]
            row = g * NW + wid

            def slab(sl, cc3):
                off = pl.multiple_of(sl * HCH, HCH)
                pltpu.sync_copy(s_hbm.at[row, pl.ds(off, HCH)], s_v)
                pltpu.sync_copy(d_hbm.at[row, pl.ds(off, HCH)], d_v)
                pltpu.sync_copy(w_hbm.at[row, pl.ds(off, HCH)], w_v)
                pltpu.async_copy(h_hbm.at[s_v.at[0]], rows0, sem0)

                def pair(j, cc2):
                    i0 = 2 * j

                    def do(i, buf, bsem, nbuf, nsem, last):
                        pltpu.make_async_copy(
                            h_hbm.at[s_v.at[i]], buf, bsem).wait()

                        @pl.when(jnp.logical_not(last))
                        def _():
                            pltpu.async_copy(
                                h_hbm.at[s_v.at[i + 1]], nbuf, nsem)
                        _scale_rows(
                            buf, w_v,
                            lambda r: [jnp.full((16,), i, _i32),
                                       jnp.full((16,), r, _i32)], C)
                        pltpu.sync_copy(buf, numer_sp.at[d_v.at[i]],
                                        add=True)

                    do(i0, rows0, sem0, rows1, sem1, False)
                    do(i0 + 1, rows1, sem1, rows0, sem0,
                       j == HCH // 2 - 1)
                    return cc2

                lax.fori_loop(0, HCH // 2, pair, 0)
                return cc3

            lax.fori_loop(0, CH // HCH, slab, 0)
            plsc.subcore_barrier()
            pltpu.sync_copy(numer_sp.at[pl.ds(base, PER_TILE)],
                            numer_out.at[c, g, pl.ds(base, PER_TILE)])
            pltpu.sync_copy(z_hbm, numer_sp.at[pl.ds(base, PER_TILE)])
            plsc.subcore_barrier()

    return pl.kernel(
        body,
        out_type=jax.ShapeDtypeStruct((NC, NG, NPAD, HID), _f32),
        mesh=_mesh,
        compiler_params=_sc_params,
        scratch_types=[
            pltpu.VMEM((HCH, C), _i32),
            pltpu.VMEM((HCH, C), _i32),
            pltpu.VMEM((HCH, C), _f32),
            pltpu.VMEM((C, HID), _f32),
            pltpu.VMEM((C, HID), _f32),
            pltpu.VMEM_SHARED((NPAD, HID), _f32),
            pltpu.SemaphoreType.DMA,
            pltpu.SemaphoreType.DMA,
        ],
    )(*h_list, w_flat, s4, d4, zrows)


CH2 = E_PAD // (NS * C)


def _sc_gcn_pass(hcat, dinv_flat, s4, d4, zrows):
    def body(*refs):
        (h_hbm, a_hbm, s_hbm, d_hbm, z_hbm, numer_out,
         s_v, d_v, a_v, w_v, rows0, rows1,
         numer_sp, sem0, sem1, scsem) = refs
        c = lax.axis_index("c")
        t = lax.axis_index("s")
        base = t * PER_TILE
        pltpu.sync_copy(z_hbm, numer_sp.at[pl.ds(base, PER_TILE)])
        pltpu.sync_copy(a_hbm.at[pl.ds(c * NPAD, TPAD)], a_v)
        plsc.subcore_barrier()
        if True:
            row = c * NS + t
            for cc in range(3):
                coff = jnp.full((16,), (c * 3 + cc) * N_NODE, _i32)

                def slab(sl, cc3):
                    off = pl.multiple_of(sl * HCH, HCH)
                    pltpu.sync_copy(s_hbm.at[row, pl.ds(off, HCH)], s_v)
                    pltpu.sync_copy(d_hbm.at[row, pl.ds(off, HCH)], d_v)

                    def wchunk(i, cc2):
                        for k in range(C // 16):
                            sv = s_v[i, pl.ds(k * 16, 16)]
                            dv = d_v[i, pl.ds(k * 16, 16)]
                            w_v[i, pl.ds(k * 16, 16)] = (
                                plsc.load_gather(a_v, [sv])
                                * plsc.load_gather(a_v, [dv]))
                            s_v[i, pl.ds(k * 16, 16)] = sv + coff
                        return cc2

                    lax.fori_loop(0, HCH, wchunk, 0)
                    pltpu.async_copy(h_hbm.at[s_v.at[0]], rows0, sem0)

                    def pair(j, cc2):
                        i0 = 2 * j

                        def do(i, buf, bsem, nbuf, nsem, last):
                            pltpu.make_async_copy(
                                h_hbm.at[s_v.at[i]], buf, bsem).wait()

                            @pl.when(jnp.logical_not(last))
                            def _():
                                pltpu.async_copy(
                                    h_hbm.at[s_v.at[i + 1]], nbuf, nsem)
                            _scale_rows(
                                buf, w_v,
                                lambda r: [jnp.full((16,), i, _i32),
                                           jnp.full((16,), r, _i32)], C)
                            pltpu.sync_copy(buf, numer_sp.at[d_v.at[i]],
                                            add=True)

                        do(i0, rows0, sem0, rows1, sem1, False)
                        do(i0 + 1, rows1, sem1, rows0, sem0,
                           j == HCH // 2 - 1)
                        return cc2

                    lax.fori_loop(0, HCH // 2, pair, 0)
                    return cc3

                lax.fori_loop(0, CH2 // HCH, slab, 0)
                plsc.subcore_barrier()
                pltpu.sync_copy(
                    numer_sp.at[pl.ds(base, PER_TILE)],
                    numer_out.at[c, cc, pl.ds(base, PER_TILE)])
                pltpu.sync_copy(z_hbm, numer_sp.at[pl.ds(base, PER_TILE)])
                plsc.subcore_barrier()

    return pl.kernel(
        body,
        out_type=jax.ShapeDtypeStruct((NC, 3, NPAD, HID), _f32),
        mesh=_mesh,
        compiler_params=_sc_params,
        scratch_types=[
            pltpu.VMEM((HCH, C), _i32),
            pltpu.VMEM((HCH, C), _i32),
            pltpu.VMEM((TPAD,), _f32),
            pltpu.VMEM((HCH, C), _f32),
            pltpu.VMEM((C, HID), _f32),
            pltpu.VMEM((C, HID), _f32),
            pltpu.VMEM_SHARED((NPAD, HID), _f32),
            pltpu.SemaphoreType.DMA,
            pltpu.SemaphoreType.DMA,
            pltpu.SemaphoreType.DMA,
        ],
    )(hcat, dinv_flat, s4, d4, zrows)


def _sc_gather(tbl, idx2):
    M = idx2.shape[0]
    NSD = M // NW

    def body(tbl_hbm, idx_hbm, out_hbm, idx_v, rows_v, sem):
        c = lax.axis_index("c")
        t = lax.axis_index("s")
        wid = c * NS + t
        for side in range(NSD):
            row = side * NW + wid
            pltpu.sync_copy(idx_hbm.at[row], idx_v)
            for g in range(IPW // 80):
                pltpu.async_copy(tbl_hbm.at[idx_v.at[pl.ds(g * 80, 80)]],
                                 rows_v, sem).wait()
                pltpu.sync_copy(rows_v,
                                out_hbm.at[pl.ds(row * IPW + g * 80, 80)])

    return pl.kernel(
        body,
        out_type=jax.ShapeDtypeStruct((M * IPW, CONCAT), _f32),
        mesh=_mesh,
        compiler_params=_sc_params,
        scratch_types=[
            pltpu.VMEM((IPW,), _i32),
            pltpu.VMEM((80, CONCAT), _f32),
            pltpu.SemaphoreType.DMA,
        ],
    )(tbl, idx2)


def _sc_deg(d4):
    NSD = d4.shape[0] // NW

    def body(d_hbm, deg_out, d_v, ones_v, z1_v, d1_v, deg_sp):
        c = lax.axis_index("c")
        t = lax.axis_index("s")
        wid = c * NS + t
        z16 = jnp.zeros((16,), _f32)
        o16 = jnp.ones((16,), _f32)
        _fill_1d(ones_v, C // 16, o16)
        _fill_1d(z1_v, PER_TILE // 16, z16)
        base = t * PER_TILE
        pltpu.sync_copy(z1_v, deg_sp.at[pl.ds(base, PER_TILE)])
        plsc.subcore_barrier()
        for side in range(NSD):
            pltpu.sync_copy(d_hbm.at[side * NW + wid], d_v)

            def chunk(i, cc):
                pltpu.sync_copy(ones_v, deg_sp.at[d_v.at[i]], add=True)
                return cc

            lax.fori_loop(0, CH, chunk, 0)
            plsc.subcore_barrier()
            pltpu.sync_copy(deg_sp.at[pl.ds(base, PER_TILE)], d1_v)
            pltpu.sync_copy(
                d1_v,
                deg_out.at[pl.ds((c * NSD + side) * NPAD + base, PER_TILE)])
            pltpu.sync_copy(z1_v, deg_sp.at[pl.ds(base, PER_TILE)])
            plsc.subcore_barrier()

    return pl.kernel(
        body,
        out_type=jax.ShapeDtypeStruct((NC * NSD * NPAD,), _f32),
        mesh=_mesh,
        compiler_params=_sc_params,
        scratch_types=[
            pltpu.VMEM((CH, C), _i32),
            pltpu.VMEM((C,), _f32),
            pltpu.VMEM((PER_TILE,), _f32),
            pltpu.VMEM((PER_TILE,), _f32),
            pltpu.VMEM_SHARED((NPAD,), _f32),
        ],
    )(d4)


def _pad_edges(s, d, shape=(NW, CH, C)):
    npad = E_PAD - s.shape[0]
    ar = jnp.arange(npad, dtype=_i32) % ND
    s3 = jnp.concatenate([s, ar]).reshape(shape)
    d3 = jnp.concatenate([d, N_NODE + ar]).reshape(shape)
    return s3, d3


def _pad_nodes(x):
    return jnp.pad(x, (0, NPAD - N_NODE))


def kernel(params, left_x, right_x, left_graph_index, right_graph_index,
           left_x_batch, right_x_batch, left_diag_cnt, right_diag_cnt,
           edges1_diag, edges2_diag, edges1_proce, edges2_proce,
           edges1_atc, edges2_atc):
    p = params
    names = ["diag", "proce", "atc"]
    edges = {"diag": (edges1_diag, edges2_diag),
             "proce": (edges1_proce, edges2_proce),
             "atc": (edges1_atc, edges2_atc)}
    zrows = jnp.zeros((PER_TILE, HID), _f32)

    Wext, bias = {}, {}
    for name in names:
        W = p["gat_" + name + "_W"]
        wsrc = W @ p["gat_" + name + "_asrc"]
        wdst = W @ p["gat_" + name + "_adst"]
        Wext[name] = jnp.concatenate([W, wsrc[:, None], wdst[:, None]],
                                     axis=1)
        bias[name] = p["gat_" + name + "_b"]

    xs = {name: p["emb_" + name] for name in names}
    for layer in range(2):
        hs_all, hd_all, h_all, sl, dl = {}, {}, [], [], []
        for name in names:
            hext = _mm(xs[name], Wext[name])
            h_all.append(hext[:, :HID])
            hs_all[name] = hext[:, HID]
            hd_all[name] = hext[:, HID + 1]
            e = edges[name][layer]
            s3, d3 = _pad_edges(e[0], e[1])
            sl.append(s3)
            dl.append(d3)
        a_flat = jnp.concatenate([_pad_nodes(hs_all[n]) for n in names])
        b_flat = jnp.concatenate([_pad_nodes(hd_all[n]) for n in names])
        s4 = jnp.concatenate(sl)
        d4 = jnp.concatenate(dl)
        w_flat, denom = _sc_gat_wpass(a_flat, b_flat, s4, d4)
        numer = _sc_gat_rows(h_all, w_flat, s4, d4, zrows)
        denom = denom.reshape(NC, 3, NPAD)
        for g, name in enumerate(names):
            nm = (numer[0, g] + numer[1, g])[:N_NODE]
            dn = (denom[0, g] + denom[1, g])[:N_NODE]
            el = hs_all[name] + hd_all[name]
            wl = jnp.exp(jnp.where(el < 0, 0.2 * el, el))
            xs[name] = ((nm + h_all[g] * wl[:, None])
                        / (dn + wl)[:, None] + bias[name])

    all_emb = jnp.concatenate([xs[n] for n in names], axis=0)
    descW = _mm(p["desc_emb"], p["Wd"], p["bd"])
    imgW = _mm(p["img_emb"], p["Wi"], p["bi"])
    x_all = jnp.concatenate([all_emb, descW, imgW], axis=1)

    npad_i = NW * IPW - N_NODE
    pad_i = jnp.arange(npad_i, dtype=_i32) % ND
    idx2 = jnp.concatenate(
        [jnp.concatenate([left_x[:, 0], pad_i]),
         jnp.concatenate([right_x[:, 0], pad_i])]).reshape(2 * NW, IPW)
    xg = _sc_gather(x_all, idx2)
    x_sides = [xg[:N_NODE], xg[NW * IPW:NW * IPW + N_NODE]]

    sL, dL = _pad_edges(left_graph_index[0], left_graph_index[1])
    sR, dR = _pad_edges(right_graph_index[0], right_graph_index[1])
    d4 = jnp.concatenate([dL, dR])
    sLg, dLg = _pad_edges(left_graph_index[0], left_graph_index[1],
                          (NS, CH2, C))
    sRg, dRg = _pad_edges(right_graph_index[0], right_graph_index[1],
                          (NS, CH2, C))
    s4g = jnp.concatenate([sLg, sRg])
    d4g = jnp.concatenate([dLg, dRg])
    deg = _sc_deg(d4).reshape(NC, 2, NPAD)
    dinvs = []
    for side in range(2):
        dg = 1.0 + (deg[0, side] + deg[1, side])[:N_NODE]
        dinvs.append(lax.rsqrt(jnp.clip(dg, 1.0, None)))
    dinv_flat = jnp.concatenate([_pad_nodes(dv) for dv in dinvs])

    for layer, (W, b) in enumerate([(p["gcn1_W"], p["gcn1_b"]),
                                    (p["gcn2_W"], p["gcn2_b"])]):
        hs = [_mm(x_sides[side], W) for side in range(2)]
        hcat = jnp.concatenate(
            [hs[side][:, cc * HID:(cc + 1) * HID]
             for side in range(2) for cc in range(3)], axis=0)
        numer = _sc_gcn_pass(hcat, dinv_flat, s4g, d4g, zrows)
        for side in range(2):
            agg = jnp.concatenate([numer[side, cc, :N_NODE]
                                   for cc in range(3)], axis=1)
            out = agg + hs[side] * (dinvs[side] * dinvs[side])[:, None] + b
            if layer == 0:
                out = jnp.where(out > 0, out, jnp.expm1(out))
            x_sides[side] = out

    return x_sides[0], x_sides[1]
